# baseline (speedup 1.0000x reference)
import functools

import jax
import jax.numpy as jnp
from jax import lax
from jax.experimental import pallas as pl
from jax.experimental.pallas import tpu as pltpu
from jax.experimental.pallas import tpu_sc as plsc

_D = 128
_L = 128
_SUB = 1
_C = _SUB * _L
_NS = 5
_NW = 32


def _emb_body(ids_hbm, table_hbm, out_hbm, idx_v, rows_v, table_sp,
              sems, *, nchunk, per_w):
  sid = lax.axis_index("s")
  wid = sid * 2 + lax.axis_index("c")
  base = wid * per_w

  @pl.when(sid == 0)
  def _stage():
    pltpu.sync_copy(table_hbm, table_sp)

  plsc.subcore_barrier()

  row_base = wid * (per_w // _L)
  gsems, wsems = sems[:_NS], sems[_NS:]

  def idx_load(c, s):
    pltpu.sync_copy(ids_hbm.at[pl.ds(row_base + c * _SUB, _SUB)], idx_v.at[s])

  def gather_start(s):
    for j in range(_SUB):
      pltpu.make_async_copy(
          table_sp.at[idx_v.at[s, j]],
          rows_v.at[s].at[pl.ds(j * _L, _L)],
          gsems[s]).start()

  def gather_wait(s):
    for j in range(_SUB):
      pltpu.make_async_copy(
          table_sp.at[idx_v.at[s, j]],
          rows_v.at[s].at[pl.ds(j * _L, _L)],
          gsems[s]).wait()

  def write_start(c, s):
    pltpu.make_async_copy(
        rows_v.at[s], out_hbm.at[pl.ds(base + c * _C, _C)], wsems[s]).start()

  def write_wait(c, s):
    pltpu.make_async_copy(
        rows_v.at[s], out_hbm.at[pl.ds(base + c * _C, _C)], wsems[s]).wait()

  for c in range(_NS):
    idx_load(c, c)
    gather_start(c)
  for c in range(_NS - 1):
    gather_wait(c)
    write_start(c, c)

  def body(g, carry):
    for d in range(_NS):
      i = _NS * g + d
      write_wait(i - _NS, d)
      idx_load(i, d)
      gather_start(d)
      gather_wait((d - 1) % _NS)
      write_start(i - 1, (d - 1) % _NS)
    return carry

  lax.fori_loop(1, nchunk // _NS, body, 0)

  last = nchunk - 1
  gather_wait(last % _NS)
  write_start(last, last % _NS)
  for k in range(_NS):
    write_wait(last - k, (last - k) % _NS)


def kernel(word_ids, n_words, table):
  del n_words
  b, h = word_ids.shape
  n = b * h
  ids = word_ids.reshape(n // _L, _L).astype(jnp.int32)
  per_w = n // _NW
  nchunk = per_w // _C
  assert nchunk % _NS == 0
  mesh = plsc.VectorSubcoreMesh(core_axis_name="c", subcore_axis_name="s")
  out = pl.kernel(
      functools.partial(_emb_body, nchunk=nchunk, per_w=per_w),
      out_type=jax.ShapeDtypeStruct((n, _D), table.dtype),
      mesh=mesh,
      scratch_types=[
          pltpu.VMEM((_NS, _SUB, _L), jnp.int32),
          pltpu.VMEM((_NS, _C, _D), jnp.float32),
          pltpu.VMEM_SHARED((table.shape[0], _D), jnp.float32),
          [pltpu.SemaphoreType.DMA] * (2 * _NS),
      ],
  )(ids, table)
  return out.reshape(b, h, _D)


# speedup vs baseline: 1.0187x; 1.0187x over previous
import functools

import jax
import jax.numpy as jnp
from jax import lax
from jax.experimental import pallas as pl
from jax.experimental.pallas import tpu as pltpu
from jax.experimental.pallas import tpu_sc as plsc

_D = 128
_L = 128
_SUB = 1
_C = _SUB * _L
_NS = 5
_NW = 32


def _emb_body(ids_hbm, table_hbm, out_hbm, idx_v, rows_v, table_sp, idx_sp,
              sems, *, nchunk, per_w):
  sid = lax.axis_index("s")
  wid = sid * 2 + lax.axis_index("c")
  base = wid * per_w

  @pl.when(sid == 0)
  def _stage():
    pltpu.sync_copy(table_hbm, table_sp)

  row_base = wid * (per_w // _L)
  pltpu.sync_copy(ids_hbm.at[pl.ds(row_base, per_w // _L)], idx_sp.at[sid])

  plsc.subcore_barrier()

  gsems, wsems = sems[:_NS], sems[_NS:]

  def idx_load(c, s):
    pltpu.sync_copy(idx_sp.at[sid].at[pl.ds(c * _SUB, _SUB)], idx_v.at[s])

  def gather_start(s):
    for j in range(_SUB):
      pltpu.make_async_copy(
          table_sp.at[idx_v.at[s, j]],
          rows_v.at[s].at[pl.ds(j * _L, _L)],
          gsems[s]).start()

  def gather_wait(s):
    for j in range(_SUB):
      pltpu.make_async_copy(
          table_sp.at[idx_v.at[s, j]],
          rows_v.at[s].at[pl.ds(j * _L, _L)],
          gsems[s]).wait()

  def write_start(c, s):
    pltpu.make_async_copy(
        rows_v.at[s], out_hbm.at[pl.ds(base + c * _C, _C)], wsems[s]).start()

  def write_wait(c, s):
    pltpu.make_async_copy(
        rows_v.at[s], out_hbm.at[pl.ds(base + c * _C, _C)], wsems[s]).wait()

  for c in range(_NS):
    idx_load(c, c)
    gather_start(c)
  for c in range(_NS - 1):
    gather_wait(c)
    write_start(c, c)

  def body(g, carry):
    for d in range(_NS):
      i = _NS * g + d
      write_wait(i - _NS, d)
      idx_load(i, d)
      gather_start(d)
      gather_wait((d - 1) % _NS)
      write_start(i - 1, (d - 1) % _NS)
    return carry

  lax.fori_loop(1, nchunk // _NS, body, 0)

  last = nchunk - 1
  gather_wait(last % _NS)
  write_start(last, last % _NS)
  for k in range(_NS):
    write_wait(last - k, (last - k) % _NS)


def kernel(word_ids, n_words, table):
  del n_words
  b, h = word_ids.shape
  n = b * h
  ids = word_ids.reshape(n // _L, _L).astype(jnp.int32)
  per_w = n // _NW
  nchunk = per_w // _C
  assert nchunk % _NS == 0
  mesh = plsc.VectorSubcoreMesh(core_axis_name="c", subcore_axis_name="s")
  out = pl.kernel(
      functools.partial(_emb_body, nchunk=nchunk, per_w=per_w),
      out_type=jax.ShapeDtypeStruct((n, _D), table.dtype),
      mesh=mesh,
      scratch_types=[
          pltpu.VMEM((_NS, _SUB, _L), jnp.int32),
          pltpu.VMEM((_NS, _C, _D), jnp.float32),
          pltpu.VMEM_SHARED((table.shape[0], _D), jnp.float32),
          pltpu.VMEM_SHARED((16, per_w // _L, _L), jnp.int32),
          [pltpu.SemaphoreType.DMA] * (2 * _NS),
      ],
  )(ids, table)
  return out.reshape(b, h, _D)
